# baseline (speedup 1.0000x reference)
import functools

import jax
import jax.numpy as jnp
from jax import lax
from jax.experimental import pallas as pl
from jax.experimental.pallas import tpu as pltpu
from jax.experimental.pallas import tpu_sc as plsc

_B = 16384
_C = 1000
_CP = 1024
_BM = 2048
_GRID = _B // _BM
_NS = 16
_PER_TILE = _B // _NS
_PR = _PER_TILE // 128


def _argmax_body(x_ref, out_ref):
    x = x_ref[...]
    m = jnp.max(x, axis=1, keepdims=True)
    idx = lax.broadcasted_iota(jnp.int32, (_BM, _C), 1)
    masked = jnp.where(x == m, idx, _C)
    pred = jnp.min(masked, axis=1)
    out_ref[...] = pred.reshape(_BM // 128, 128)


def _sc_hist_body(pred_hbm, lab_hbm, zeros_hbm, out_hbm,
                  pred_v, lab_v, hpp, htp, part_pp, part_tp,
                  red_pp, red_tp, ov, sh_pp, sh_tp):
    cid = lax.axis_index("c")
    sid = lax.axis_index("s")

    pltpu.sync_copy(pred_hbm.at[pl.ds(sid * _PR, _PR), :], pred_v)
    pltpu.sync_copy(lab_hbm.at[pl.ds(sid * _PER_TILE, _PER_TILE)], lab_v)
    pltpu.sync_copy(zeros_hbm, hpp)
    pltpu.sync_copy(zeros_hbm, htp)

    lane_base = lax.iota(jnp.int32, 16) * _CP
    ones = jnp.ones((16,), jnp.float32)

    def _scat(j, _):
        r = j // 8
        k = j % 8
        p = pred_v[r, pl.ds(k * 16, 16)]
        l = lab_v[pl.ds(j * 16, 16)]
        corr = jnp.where(p == l, 1.0, 0.0).astype(jnp.float32)
        addr = lane_base + p
        plsc.addupdate_scatter(hpp, [addr], ones)
        plsc.addupdate_scatter(htp, [addr], corr)
        return 0

    lax.fori_loop(0, _PER_TILE // 16, _scat, 0)

    def _lred(k, _):
        def _inner(l, acc):
            app, atp = acc
            return (app + hpp[pl.ds(l * _CP + k * 16, 16)],
                    atp + htp[pl.ds(l * _CP + k * 16, 16)])

        app, atp = lax.fori_loop(
            0, 16, _inner,
            (jnp.zeros((16,), jnp.float32), jnp.zeros((16,), jnp.float32)))
        part_pp[pl.ds(k * 16, 16)] = app
        part_tp[pl.ds(k * 16, 16)] = atp
        return 0

    lax.fori_loop(0, _CP // 16, _lred, 0)

    pltpu.sync_copy(part_pp, sh_pp.at[pl.ds(sid * _CP, _CP)])
    pltpu.sync_copy(part_tp, sh_tp.at[pl.ds(sid * _CP, _CP)])
    plsc.subcore_barrier()

    @pl.when(jnp.logical_and(cid == 0, sid == 0))
    def _fini():
        pltpu.sync_copy(sh_pp, red_pp)
        pltpu.sync_copy(sh_tp, red_tp)

        def _prec(k, psum):
            def _inner(t, acc):
                app, atp = acc
                return (app + red_pp[pl.ds(t * _CP + k * 16, 16)],
                        atp + red_tp[pl.ds(t * _CP + k * 16, 16)])

            pp, tp = lax.fori_loop(
                0, _NS, _inner,
                (jnp.zeros((16,), jnp.float32),
                 jnp.zeros((16,), jnp.float32)))
            safe = jnp.where(pp > 0, pp, 1.0)
            return psum + jnp.where(pp > 0, tp / safe, 0.0)

        psum = lax.fori_loop(0, _CP // 16, _prec,
                             jnp.zeros((16,), jnp.float32))
        total = jnp.sum(psum)
        ov[...] = jnp.full((16,), total, jnp.float32) * jnp.float32(1.0 / _C)
        pltpu.sync_copy(ov, out_hbm)


@functools.partial(
    pl.kernel,
    out_type=jax.ShapeDtypeStruct((16,), jnp.float32),
    mesh=plsc.VectorSubcoreMesh(core_axis_name="c", subcore_axis_name="s"),
    compiler_params=pltpu.CompilerParams(needs_layout_passes=False),
    scratch_types=[
        pltpu.VMEM((_PR, 128), jnp.int32),
        pltpu.VMEM((_PER_TILE,), jnp.int32),
        pltpu.VMEM((16 * _CP,), jnp.float32),
        pltpu.VMEM((16 * _CP,), jnp.float32),
        pltpu.VMEM((_CP,), jnp.float32),
        pltpu.VMEM((_CP,), jnp.float32),
        pltpu.VMEM((_NS * _CP,), jnp.float32),
        pltpu.VMEM((_NS * _CP,), jnp.float32),
        pltpu.VMEM((16,), jnp.float32),
        pltpu.VMEM_SHARED((_NS * _CP,), jnp.float32),
        pltpu.VMEM_SHARED((_NS * _CP,), jnp.float32),
    ],
)
def _sc_hist(pred_hbm, lab_hbm, zeros_hbm, out_hbm, *scratch):
    _sc_hist_body(pred_hbm, lab_hbm, zeros_hbm, out_hbm, *scratch)


def kernel(logits, labels):
    pred2d = pl.pallas_call(
        _argmax_body,
        grid=(_GRID,),
        in_specs=[pl.BlockSpec((_BM, _C), lambda i: (i, 0))],
        out_specs=pl.BlockSpec((_BM // 128, 128), lambda i: (i, 0)),
        out_shape=jax.ShapeDtypeStruct((_B // 128, 128), jnp.int32),
    )(logits)
    zeros = jnp.zeros((16 * _CP,), jnp.float32)
    out16 = _sc_hist(pred2d, labels, zeros)
    return out16[0].reshape(())


# speedup vs baseline: 1.0492x; 1.0492x over previous
import functools

import jax
import jax.numpy as jnp
from jax import lax
from jax.experimental import pallas as pl
from jax.experimental.pallas import tpu as pltpu
from jax.experimental.pallas import tpu_sc as plsc

_B = 16384
_C = 1000
_CP = 1024
_BM = 2048
_GRID = _B // _BM
_NS = 16
_PER_TILE = _B // _NS
_PR = _PER_TILE // 128


def _argmax_body(x_ref, out_ref):
    x = x_ref[...]
    m = jnp.max(x, axis=1, keepdims=True)
    idx = lax.broadcasted_iota(jnp.int32, (_BM, _C), 1)
    masked = jnp.where(x == m, idx, _C)
    pred = jnp.min(masked, axis=1)
    out_ref[...] = pred.reshape(_BM // 128, 128)


def _sc_hist_body(pred_hbm, lab_hbm, zi_hbm, out_hbm,
                  pred_v, lab_v, hpp, htp, part_pp, part_tp,
                  zrow, app2, atp2, ov, row0, sh_pp, sh_tp):
    cid = lax.axis_index("c")
    sid = lax.axis_index("s")

    pltpu.sync_copy(zi_hbm, row0)
    pltpu.sync_copy(pred_hbm.at[pl.ds(sid * _PR, _PR), :], pred_v)
    pltpu.sync_copy(lab_hbm.at[pl.ds(sid * _PER_TILE, _PER_TILE)], lab_v)

    z16 = jnp.zeros((16,), jnp.float32)

    def _zero(j, _):
        for u in range(16):
            hpp[pl.ds(j * 256 + u * 16, 16)] = z16
            htp[pl.ds(j * 256 + u * 16, 16)] = z16
        return 0

    lax.fori_loop(0, 16 * _CP // 256, _zero, 0)

    @pl.when(sid == 0)
    def _zero_shared():
        def _zr(j, _):
            for u in range(8):
                zrow[0, pl.ds(j * 128 + u * 16, 16)] = z16
            return 0

        lax.fori_loop(0, _CP // 128, _zr, 0)
        pltpu.sync_copy(zrow, sh_pp)
        pltpu.sync_copy(zrow, sh_tp)

    lane_base = lax.iota(jnp.int32, 16) * _CP
    ones = jnp.ones((16,), jnp.float32)

    def _scat(j, _):
        for u in range(8):
            g = j * 8 + u
            p = pred_v[g // 8, pl.ds((g % 8) * 16, 16)]
            l = lab_v[pl.ds(g * 16, 16)]
            corr = jnp.where(p == l, 1.0, 0.0).astype(jnp.float32)
            addr = lane_base + p
            plsc.addupdate_scatter(hpp, [addr], ones)
            plsc.addupdate_scatter(htp, [addr], corr)
        return 0

    lax.fori_loop(0, _PER_TILE // 16 // 8, _scat, 0)

    def _lred(k, _):
        app = z16
        atp = z16
        for l in range(16):
            app = app + hpp[pl.ds(l * _CP + k * 16, 16)]
            atp = atp + htp[pl.ds(l * _CP + k * 16, 16)]
        part_pp[0, pl.ds(k * 16, 16)] = app
        part_tp[0, pl.ds(k * 16, 16)] = atp
        return 0

    lax.fori_loop(0, _CP // 16, _lred, 0)

    plsc.subcore_barrier()

    pltpu.sync_copy(part_pp, sh_pp.at[row0], add=True)
    pltpu.sync_copy(part_tp, sh_tp.at[row0], add=True)

    plsc.subcore_barrier()

    @pl.when(jnp.logical_and(cid == 0, sid == 0))
    def _fini():
        pltpu.sync_copy(sh_pp, app2)
        pltpu.sync_copy(sh_tp, atp2)

        def _prec(k, psum):
            pp = app2[0, pl.ds(k * 16, 16)]
            tp = atp2[0, pl.ds(k * 16, 16)]
            safe = jnp.where(pp > 0, pp, 1.0)
            return psum + jnp.where(pp > 0, tp / safe, 0.0)

        psum = lax.fori_loop(0, _CP // 16, _prec, z16)
        total = jnp.sum(psum)
        ov[...] = jnp.full((16,), total, jnp.float32) * jnp.float32(1.0 / _C)
        pltpu.sync_copy(ov, out_hbm)


@functools.partial(
    pl.kernel,
    out_type=jax.ShapeDtypeStruct((16,), jnp.float32),
    mesh=plsc.VectorSubcoreMesh(core_axis_name="c", subcore_axis_name="s"),
    compiler_params=pltpu.CompilerParams(needs_layout_passes=False),
    scratch_types=[
        pltpu.VMEM((_PR, 128), jnp.int32),
        pltpu.VMEM((_PER_TILE,), jnp.int32),
        pltpu.VMEM((16 * _CP,), jnp.float32),
        pltpu.VMEM((16 * _CP,), jnp.float32),
        pltpu.VMEM((1, _CP), jnp.float32),
        pltpu.VMEM((1, _CP), jnp.float32),
        pltpu.VMEM((1, _CP), jnp.float32),
        pltpu.VMEM((1, _CP), jnp.float32),
        pltpu.VMEM((1, _CP), jnp.float32),
        pltpu.VMEM((16,), jnp.float32),
        pltpu.VMEM((1,), jnp.int32),
        pltpu.VMEM_SHARED((1, _CP), jnp.float32),
        pltpu.VMEM_SHARED((1, _CP), jnp.float32),
    ],
)
def _sc_hist(pred_hbm, lab_hbm, zi_hbm, out_hbm, *scratch):
    _sc_hist_body(pred_hbm, lab_hbm, zi_hbm, out_hbm, *scratch)


def kernel(logits, labels):
    pred2d = pl.pallas_call(
        _argmax_body,
        grid=(_GRID,),
        in_specs=[pl.BlockSpec((_BM, _C), lambda i: (i, 0))],
        out_specs=pl.BlockSpec((_BM // 128, 128), lambda i: (i, 0)),
        out_shape=jax.ShapeDtypeStruct((_B // 128, 128), jnp.int32),
    )(logits)
    out16 = _sc_hist(pred2d, labels, jnp.zeros((1,), jnp.int32))
    return out16[0].reshape(())
